# Initial kernel scaffold; baseline (speedup 1.0000x reference)
import jax
import jax.numpy as jnp
from jax.experimental import pallas as pl


def kernel(histograms, W, b, gumbel):
    raise NotImplementedError("write your pallas kernel here")


# speedup vs baseline: 1.1476x; 1.1476x over previous
import functools

import jax
import jax.numpy as jnp
from jax import lax
from jax.experimental import pallas as pl
from jax.experimental.pallas import tpu as pltpu
from jax.experimental.pallas import tpu_sc as plsc

B = 16384
D = 2048
E = 4
NC = 2
NS = 16
L = 16
NW = NC * NS
RPW = B // NW
RCHUNK = 16
NT = RPW // RCHUNK
CPR = D // L


def _sc_body(hist_hbm, w_hbm, gb_hbm, out_hbm, w_v, gb_v, x_v, out_v):
    wid = lax.axis_index("s") * NC + lax.axis_index("c")
    base = wid * RPW
    pltpu.sync_copy(w_hbm, w_v)
    pltpu.sync_copy(gb_hbm.at[pl.ds(base * E, RPW * E)], gb_v.at[pl.ds(0, RPW * E)])
    iota = lax.iota(jnp.int32, L)
    im1 = iota - 1

    def _bf16_rne(v):
        vi = plsc.bitcast(v, jnp.uint32)
        vi = ((vi + jnp.uint32(0x7FFF) + ((vi >> 16) & jnp.uint32(1)))
              & jnp.uint32(0xFFFF0000))
        return plsc.bitcast(vi, jnp.float32)

    def wr_body(i, _):
        o = i * L
        w_v[pl.ds(o, L)] = _bf16_rne(w_v[pl.ds(o, L)])
        return 0

    lax.fori_loop(0, E * D // L, wr_body, 0)

    def row_body(t, r, vacc):
        rvec = jnp.full((L,), r, jnp.int32)
        zv = jnp.zeros((L,), jnp.float32)

        def chunk_body(c, carry):
            s, u, p, l0, l1, l2, l3 = carry
            o = c * L
            x = x_v[r, pl.ds(o, L)]
            xp = plsc.load_gather(x_v, [rvec, jnp.maximum(o + im1, 0)])
            s = s + x
            u = u + jnp.where(x != 0.0, 1.0, 0.0)
            p = p + jnp.where(x != xp, 1.0, 0.0)
            xb = _bf16_rne(x)
            l0 = l0 + xb * w_v[pl.ds(o, L)]
            l1 = l1 + xb * w_v[pl.ds(D + o, L)]
            l2 = l2 + xb * w_v[pl.ds(2 * D + o, L)]
            l3 = l3 + xb * w_v[pl.ds(3 * D + o, L)]
            return s, u, p, l0, l1, l2, l3

        s, u, p, l0, l1, l2, l3 = lax.fori_loop(
            0, CPR, chunk_body, (zv, zv, zv, zv, zv, zv, zv))

        ri = t * RCHUNK + r
        s_t = jnp.sum(s)
        u_t = jnp.sum(u)
        p_t = jnp.sum(p)
        gbv = plsc.load_gather(gb_v, [ri * E + iota])
        z0 = jnp.sum(l0) + gbv[0]
        z1 = jnp.sum(l1) + gbv[1]
        z2 = jnp.sum(l2) + gbv[2]
        z3 = jnp.sum(l3) + gbv[3]
        bst, bi = z0, jnp.int32(0)
        c1 = z1 > bst
        bst, bi = jnp.where(c1, z1, bst), jnp.where(c1, 1, bi)
        c2 = z2 > bst
        bst, bi = jnp.where(c2, z2, bst), jnp.where(c2, 2, bi)
        c3 = z3 > bst
        bi = jnp.where(c3, 3, bi)
        last = x_v[r, pl.ds(D - L, L)][L - 1]
        freq = (jnp.full((L,), last) / jnp.full((L,), s_t + 1e-6))[0]
        val = jnp.where(bi == 0, s_t,
                        jnp.where(bi == 1, freq,
                                  jnp.where(bi == 2, u_t, p_t)))
        return jnp.where(iota == r, val, vacc)

    def stage_body(t, _):
        pltpu.sync_copy(hist_hbm.at[pl.ds(base + t * RCHUNK, RCHUNK)], x_v)
        vacc = lax.fori_loop(0, RCHUNK, functools.partial(row_body, t),
                             jnp.zeros((L,), jnp.float32))
        out_v[pl.ds(t * RCHUNK, RCHUNK)] = vacc
        return 0

    lax.fori_loop(0, NT, stage_body, 0)
    pltpu.sync_copy(out_v, out_hbm.at[pl.ds(base, RPW)])


_sc_kernel = pl.kernel(
    _sc_body,
    out_type=jax.ShapeDtypeStruct((B,), jnp.float32),
    mesh=plsc.VectorSubcoreMesh(core_axis_name="c", subcore_axis_name="s"),
    compiler_params=pltpu.CompilerParams(
        use_tc_tiling_on_sc=False, needs_layout_passes=False),
    scratch_types=[
        pltpu.VMEM((E * D,), jnp.float32),
        pltpu.VMEM((RPW * E + L,), jnp.float32),
        pltpu.VMEM((RCHUNK, D), jnp.float32),
        pltpu.VMEM((RPW,), jnp.float32),
    ],
)


def kernel(histograms, W, b, gumbel):
    gb = (gumbel + b[None, :]).reshape(B * E)
    return _sc_kernel(histograms, W.reshape(E * D), gb)


# speedup vs baseline: 1.5944x; 1.3893x over previous
import functools

import jax
import jax.numpy as jnp
from jax import lax
from jax.experimental import pallas as pl
from jax.experimental.pallas import tpu as pltpu
from jax.experimental.pallas import tpu_sc as plsc

B = 16384
D = 2048
E = 4
NC = 2
NS = 16
L = 16
NW = NC * NS
RPW = B // NW
RCHUNK = 16
NT = RPW // RCHUNK
CPR = D // L


def _bf16_rne(v):
    vi = plsc.bitcast(v, jnp.uint32)
    vi = ((vi + jnp.uint32(0x7FFF) + ((vi >> 16) & jnp.uint32(1)))
          & jnp.uint32(0xFFFF0000))
    return plsc.bitcast(vi, jnp.float32)


def _sc_body(hist_hbm, w_hbm, gb_hbm, out_hbm, w_v, gb_v, x_v, out_v, sem):
    wid = lax.axis_index("s") * NC + lax.axis_index("c")
    base = wid * RPW
    pltpu.sync_copy(w_hbm, w_v)
    pltpu.sync_copy(gb_hbm.at[pl.ds(base * E, RPW * E)],
                    gb_v.at[pl.ds(0, RPW * E)])
    iota = lax.iota(jnp.int32, L)

    def wr_body(i, _):
        o = i * L
        w_v[pl.ds(o, L)] = _bf16_rne(w_v[pl.ds(o, L)])
        return 0

    lax.fori_loop(0, E * D // L, wr_body, 0)

    im1c = jnp.maximum(iota - 1, 0)

    def row_body(t, r, vacc):
        xr = t % 2 * RCHUNK + r
        rvec = jnp.full((L,), xr, jnp.int32)

        x0 = x_v[xr, pl.ds(0, L)]
        xp0 = plsc.load_gather(x_v, [rvec, im1c])
        xb0 = _bf16_rne(x0)
        init = (x0,
                jnp.where(x0 != 0.0, 1.0, 0.0),
                jnp.where(x0 != xp0, 1.0, 0.0),
                xb0 * w_v[pl.ds(0, L)],
                xb0 * w_v[pl.ds(D, L)],
                xb0 * w_v[pl.ds(2 * D, L)],
                xb0 * w_v[pl.ds(3 * D, L)])

        def chunk_body(c, carry):
            s, u, p, l0, l1, l2, l3 = carry
            o = c * L
            x = x_v[xr, pl.ds(o, L)]
            xp = x_v[xr, pl.ds(o - 1, L)]
            s = s + x
            u = u + jnp.where(x != 0.0, 1.0, 0.0)
            p = p + jnp.where(x != xp, 1.0, 0.0)
            xb = _bf16_rne(x)
            l0 = l0 + xb * w_v[pl.ds(o, L)]
            l1 = l1 + xb * w_v[pl.ds(D + o, L)]
            l2 = l2 + xb * w_v[pl.ds(2 * D + o, L)]
            l3 = l3 + xb * w_v[pl.ds(3 * D + o, L)]
            return s, u, p, l0, l1, l2, l3

        s, u, p, l0, l1, l2, l3 = lax.fori_loop(
            1, CPR, chunk_body, init, unroll=4)

        ri = t * RCHUNK + r
        s_t = jnp.sum(s)
        u_t = jnp.sum(u)
        p_t = jnp.sum(p)
        gbv = plsc.load_gather(gb_v, [ri * E + iota])
        z0 = jnp.sum(l0) + gbv[0]
        z1 = jnp.sum(l1) + gbv[1]
        z2 = jnp.sum(l2) + gbv[2]
        z3 = jnp.sum(l3) + gbv[3]
        bst, bi = z0, jnp.int32(0)
        c1 = z1 > bst
        bst, bi = jnp.where(c1, z1, bst), jnp.where(c1, 1, bi)
        c2 = z2 > bst
        bst, bi = jnp.where(c2, z2, bst), jnp.where(c2, 2, bi)
        c3 = z3 > bst
        bi = jnp.where(c3, 3, bi)
        last = x_v[xr, pl.ds(D - L, L)][L - 1]
        freq = (jnp.full((L,), last) / jnp.full((L,), s_t + 1e-6))[0]
        val = jnp.where(bi == 0, s_t,
                        jnp.where(bi == 1, freq,
                                  jnp.where(bi == 2, u_t, p_t)))
        return jnp.where(iota == r, val, vacc)

    def start_copy(t):
        return pltpu.async_copy(
            hist_hbm.at[pl.ds(base + t * RCHUNK, RCHUNK)],
            x_v.at[pl.ds(t % 2 * RCHUNK, RCHUNK)], sem)

    def wait_copy(t):
        pltpu.make_async_copy(
            hist_hbm.at[pl.ds(base + t * RCHUNK, RCHUNK)],
            x_v.at[pl.ds(t % 2 * RCHUNK, RCHUNK)], sem).wait()

    start_copy(0)

    def stage_body(t, _):
        wait_copy(t)

        @pl.when(t + 1 < NT)
        def _():
            start_copy(t + 1)

        vacc = lax.fori_loop(0, RCHUNK, functools.partial(row_body, t),
                             jnp.zeros((L,), jnp.float32))
        out_v[pl.ds(t * RCHUNK, RCHUNK)] = vacc
        return 0

    lax.fori_loop(0, NT, stage_body, 0)
    pltpu.sync_copy(out_v, out_hbm.at[pl.ds(base, RPW)])


_sc_kernel = pl.kernel(
    _sc_body,
    out_type=jax.ShapeDtypeStruct((B,), jnp.float32),
    mesh=plsc.VectorSubcoreMesh(core_axis_name="c", subcore_axis_name="s"),
    compiler_params=pltpu.CompilerParams(
        use_tc_tiling_on_sc=False, needs_layout_passes=False),
    scratch_types=[
        pltpu.VMEM((E * D,), jnp.float32),
        pltpu.VMEM((RPW * E + L,), jnp.float32),
        pltpu.VMEM((2 * RCHUNK, D), jnp.float32),
        pltpu.VMEM((RPW,), jnp.float32),
        pltpu.SemaphoreType.DMA,
    ],
)


def kernel(histograms, W, b, gumbel):
    gb = (gumbel + b[None, :]).reshape(B * E)
    return _sc_kernel(histograms, W.reshape(E * D), gb)


# speedup vs baseline: 2.7105x; 1.7001x over previous
import functools

import jax
import jax.numpy as jnp
from jax import lax
from jax.experimental import pallas as pl
from jax.experimental.pallas import tpu as pltpu
from jax.experimental.pallas import tpu_sc as plsc

B = 16384
D = 2048
E = 4
NC = 2
NS = 16
L = 16
NW = NC * NS
RPW = B // NW
RCHUNK = 16
NT = RPW // RCHUNK
CPR = D // L
MMB = 1024
CMB = 2048


def _sc_body(hist_hbm, out_hbm, x_v, out_v, sem):
    wid = lax.axis_index("s") * NC + lax.axis_index("c")
    base = wid * RPW
    iota = lax.iota(jnp.int32, L)
    im1c = jnp.maximum(iota - 1, 0)

    def row_body(t, r, vaccs):
        vs, vf, vu, vp = vaccs
        xr = t % 2 * RCHUNK + r
        rvec = jnp.full((L,), xr, jnp.int32)

        x0 = x_v[xr, pl.ds(0, L)]
        xp0 = plsc.load_gather(x_v, [rvec, im1c])
        xi0 = plsc.bitcast(x0, jnp.int32)
        init = (x0,
                plsc.all_reduce_population_count(xi0 == 0),
                plsc.all_reduce_population_count(
                    xi0 != plsc.bitcast(xp0, jnp.int32)))

        def chunk_body(c, carry):
            s, zc, pc = carry
            o = c * L
            x = x_v[xr, pl.ds(o, L)]
            xp = x_v[xr, pl.ds(o - 1, L)]
            xi = plsc.bitcast(x, jnp.int32)
            s = s + x
            zc = zc + plsc.all_reduce_population_count(xi == 0)
            pc = pc + plsc.all_reduce_population_count(
                xi != plsc.bitcast(xp, jnp.int32))
            return s, zc, pc

        s, zc, pc = lax.fori_loop(1, CPR, chunk_body, init, unroll=4)

        s_t = jnp.sum(s)
        u_t = (jnp.full((L,), D) - zc).astype(jnp.float32)[0]
        p_t = pc.astype(jnp.float32)[0]
        last = x_v[xr, pl.ds(D - L, L)][L - 1]
        freq = (jnp.full((L,), last) / jnp.full((L,), s_t + 1e-6))[0]
        ins = iota == r
        return (jnp.where(ins, s_t, vs), jnp.where(ins, freq, vf),
                jnp.where(ins, u_t, vu), jnp.where(ins, p_t, vp))

    def start_copy(t):
        return pltpu.async_copy(
            hist_hbm.at[pl.ds(base + t * RCHUNK, RCHUNK)],
            x_v.at[pl.ds(t % 2 * RCHUNK, RCHUNK)], sem)

    def wait_copy(t):
        pltpu.make_async_copy(
            hist_hbm.at[pl.ds(base + t * RCHUNK, RCHUNK)],
            x_v.at[pl.ds(t % 2 * RCHUNK, RCHUNK)], sem).wait()

    start_copy(0)

    def stage_body(t, _):
        wait_copy(t)

        @pl.when(t + 1 < NT)
        def _():
            start_copy(t + 1)

        z = jnp.zeros((L,), jnp.float32)
        vs, vf, vu, vp = lax.fori_loop(
            0, RCHUNK, functools.partial(row_body, t), (z, z, z, z))
        out_v[0, pl.ds(t * RCHUNK, RCHUNK)] = vs
        out_v[1, pl.ds(t * RCHUNK, RCHUNK)] = vf
        out_v[2, pl.ds(t * RCHUNK, RCHUNK)] = vu
        out_v[3, pl.ds(t * RCHUNK, RCHUNK)] = vp
        return 0

    lax.fori_loop(0, NT, stage_body, 0)
    pltpu.sync_copy(out_v, out_hbm.at[:, pl.ds(base, RPW)])


_sc_stats = pl.kernel(
    _sc_body,
    out_type=jax.ShapeDtypeStruct((E, B), jnp.float32),
    mesh=plsc.VectorSubcoreMesh(core_axis_name="c", subcore_axis_name="s"),
    compiler_params=pltpu.CompilerParams(
        use_tc_tiling_on_sc=False, needs_layout_passes=False),
    scratch_types=[
        pltpu.VMEM((2 * RCHUNK, D), jnp.float32),
        pltpu.VMEM((E, RPW), jnp.float32),
        pltpu.SemaphoreType.DMA,
    ],
)


def _mm_body(x_ref, w_ref, gb_ref, o_ref):
    xb = x_ref[...].astype(jnp.bfloat16)
    wb = w_ref[...].astype(jnp.bfloat16)
    z = lax.dot_general(xb, wb, (((1,), (0,)), ((), ())),
                        preferred_element_type=jnp.float32)
    o_ref[...] = z + gb_ref[...]


_mm = pl.pallas_call(
    _mm_body,
    grid=(B // MMB,),
    in_specs=[
        pl.BlockSpec((MMB, D), lambda i: (i, 0)),
        pl.BlockSpec((D, E), lambda i: (0, 0)),
        pl.BlockSpec((MMB, E), lambda i: (i, 0)),
    ],
    out_specs=pl.BlockSpec((MMB, E), lambda i: (i, 0)),
    out_shape=jax.ShapeDtypeStruct((B, E), jnp.float32),
)


def _comb_body(zg_ref, st_ref, o_ref):
    z0 = zg_ref[:, 0]
    z1 = zg_ref[:, 1]
    z2 = zg_ref[:, 2]
    z3 = zg_ref[:, 3]
    bst, bi = z0, jnp.zeros_like(z0, dtype=jnp.int32)
    c1 = z1 > bst
    bst, bi = jnp.where(c1, z1, bst), jnp.where(c1, 1, bi)
    c2 = z2 > bst
    bst, bi = jnp.where(c2, z2, bst), jnp.where(c2, 2, bi)
    c3 = z3 > bst
    bi = jnp.where(c3, 3, bi)
    val = jnp.where(bi == 0, st_ref[0, :],
                    jnp.where(bi == 1, st_ref[1, :],
                              jnp.where(bi == 2, st_ref[2, :], st_ref[3, :])))
    o_ref[...] = val


_comb = pl.pallas_call(
    _comb_body,
    grid=(B // CMB,),
    in_specs=[
        pl.BlockSpec((CMB, E), lambda i: (i, 0)),
        pl.BlockSpec((E, CMB), lambda i: (0, i)),
    ],
    out_specs=pl.BlockSpec((CMB,), lambda i: (i,)),
    out_shape=jax.ShapeDtypeStruct((B,), jnp.float32),
)


def kernel(histograms, W, b, gumbel):
    gb = gumbel + b[None, :]
    stats = _sc_stats(histograms)
    zg = _mm(histograms, W.T, gb)
    return _comb(zg, stats)


# speedup vs baseline: 3.2329x; 1.1927x over previous
import functools

import jax
import jax.numpy as jnp
from jax import lax
from jax.experimental import pallas as pl
from jax.experimental.pallas import tpu as pltpu
from jax.experimental.pallas import tpu_sc as plsc

B = 16384
D = 2048
E = 4
NC = 2
NS = 16
L = 16
NW = NC * NS
RPW = B // NW
RCHUNK = 16
NT = RPW // RCHUNK
CPR = D // L
MMB = 1024
CMB = 2048


def _sc_body(hist_hbm, out_hbm, x_v, out_v, sem):
    wid = lax.axis_index("s") * NC + lax.axis_index("c")
    base = wid * RPW
    iota = lax.iota(jnp.int32, L)
    im1c = jnp.maximum(iota - 1, 0)

    def row_body(t, r, vaccs):
        vs, vf, vu, vp = vaccs
        xr = t % 2 * RCHUNK + r
        rvec = jnp.full((L,), xr, jnp.int32)

        x0 = x_v[xr, pl.ds(0, L)]
        xp0 = plsc.load_gather(x_v, [rvec, im1c])
        xi0 = plsc.bitcast(x0, jnp.int32)
        init = (x0,
                plsc.all_reduce_population_count(xi0 == 0),
                plsc.all_reduce_population_count(
                    xi0 != plsc.bitcast(xp0, jnp.int32)))

        def chunk_body(c, carry):
            s, zc, pc = carry
            o = c * L
            x = x_v[xr, pl.ds(o, L)]
            xp = x_v[xr, pl.ds(o - 1, L)]
            xi = plsc.bitcast(x, jnp.int32)
            s = s + x
            zc = zc + plsc.all_reduce_population_count(xi == 0)
            pc = pc + plsc.all_reduce_population_count(
                xi != plsc.bitcast(xp, jnp.int32))
            return s, zc, pc

        s, zc, pc = lax.fori_loop(1, CPR, chunk_body, init, unroll=4)

        s_t = jnp.sum(s)
        u_t = (jnp.full((L,), D) - zc).astype(jnp.float32)[0]
        p_t = pc.astype(jnp.float32)[0]
        last = x_v[xr, pl.ds(D - L, L)][L - 1]
        freq = (jnp.full((L,), last) / jnp.full((L,), s_t + 1e-6))[0]
        ins = iota == r
        return (jnp.where(ins, s_t, vs), jnp.where(ins, freq, vf),
                jnp.where(ins, u_t, vu), jnp.where(ins, p_t, vp))

    def start_copy(t):
        return pltpu.async_copy(
            hist_hbm.at[pl.ds(base + t * RCHUNK, RCHUNK)],
            x_v.at[pl.ds(t % 2 * RCHUNK, RCHUNK)], sem)

    def wait_copy(t):
        pltpu.make_async_copy(
            hist_hbm.at[pl.ds(base + t * RCHUNK, RCHUNK)],
            x_v.at[pl.ds(t % 2 * RCHUNK, RCHUNK)], sem).wait()

    start_copy(0)

    def stage_body(t, _):
        wait_copy(t)

        @pl.when(t + 1 < NT)
        def _():
            start_copy(t + 1)

        z = jnp.zeros((L,), jnp.float32)
        vs, vf, vu, vp = lax.fori_loop(
            0, RCHUNK, functools.partial(row_body, t), (z, z, z, z))
        out_v[0, pl.ds(t * RCHUNK, RCHUNK)] = vs
        out_v[1, pl.ds(t * RCHUNK, RCHUNK)] = vf
        out_v[2, pl.ds(t * RCHUNK, RCHUNK)] = vu
        out_v[3, pl.ds(t * RCHUNK, RCHUNK)] = vp
        return 0

    lax.fori_loop(0, NT, stage_body, 0)
    pltpu.sync_copy(out_v, out_hbm.at[:, pl.ds(base, RPW)])


_sc_stats = pl.kernel(
    _sc_body,
    out_type=jax.ShapeDtypeStruct((E, B), jnp.float32),
    mesh=plsc.VectorSubcoreMesh(core_axis_name="c", subcore_axis_name="s"),
    compiler_params=pltpu.CompilerParams(
        use_tc_tiling_on_sc=True, needs_layout_passes=False),
    scratch_types=[
        pltpu.VMEM((2 * RCHUNK, D), jnp.float32),
        pltpu.VMEM((E, RPW), jnp.float32),
        pltpu.SemaphoreType.DMA,
    ],
)


def _mm_body(x_ref, w_ref, gb_ref, o_ref):
    xb = x_ref[...].astype(jnp.bfloat16)
    wb = w_ref[...].astype(jnp.bfloat16)
    z = lax.dot_general(xb, wb, (((1,), (0,)), ((), ())),
                        preferred_element_type=jnp.float32)
    o_ref[...] = z + gb_ref[...]


_mm = pl.pallas_call(
    _mm_body,
    grid=(B // MMB,),
    in_specs=[
        pl.BlockSpec((MMB, D), lambda i: (i, 0)),
        pl.BlockSpec((D, E), lambda i: (0, 0)),
        pl.BlockSpec((MMB, E), lambda i: (i, 0)),
    ],
    out_specs=pl.BlockSpec((MMB, E), lambda i: (i, 0)),
    out_shape=jax.ShapeDtypeStruct((B, E), jnp.float32),
)


def _comb_body(zg_ref, st_ref, o_ref):
    z0 = zg_ref[:, 0]
    z1 = zg_ref[:, 1]
    z2 = zg_ref[:, 2]
    z3 = zg_ref[:, 3]
    bst, bi = z0, jnp.zeros_like(z0, dtype=jnp.int32)
    c1 = z1 > bst
    bst, bi = jnp.where(c1, z1, bst), jnp.where(c1, 1, bi)
    c2 = z2 > bst
    bst, bi = jnp.where(c2, z2, bst), jnp.where(c2, 2, bi)
    c3 = z3 > bst
    bi = jnp.where(c3, 3, bi)
    val = jnp.where(bi == 0, st_ref[0, :],
                    jnp.where(bi == 1, st_ref[1, :],
                              jnp.where(bi == 2, st_ref[2, :], st_ref[3, :])))
    o_ref[...] = val


_comb = pl.pallas_call(
    _comb_body,
    grid=(B // CMB,),
    in_specs=[
        pl.BlockSpec((CMB, E), lambda i: (i, 0)),
        pl.BlockSpec((E, CMB), lambda i: (0, i)),
    ],
    out_specs=pl.BlockSpec((CMB,), lambda i: (i,)),
    out_shape=jax.ShapeDtypeStruct((B,), jnp.float32),
)


def kernel(histograms, W, b, gumbel):
    gb = gumbel + b[None, :]
    stats = _sc_stats(histograms)
    zg = _mm(histograms, W.T, gb)
    return _comb(zg, stats)
